# baseline (speedup 1.0000x reference)
import jax
import jax.numpy as jnp
from jax import lax
from jax.experimental import pallas as pl
from jax.experimental.pallas import tpu as pltpu
from jax.experimental.pallas import tpu_sc as plsc

N = 10000
E = 320000
D = 128
DE = 4
C = 2

NC = 2
NS = 16
NW = NC * NS
CH = 128
WCH = 80
PAIRS = WCH // 2
EPAD = NW * WCH * CH
N8 = N + 8

ROWS_PER_TILE = 624
TAIL_ROWS = N8 - NS * ROWS_PER_TILE
TAIL_OFF = NS * ROWS_PER_TILE

_mesh = plsc.VectorSubcoreMesh(
    core_axis_name="c", subcore_axis_name="s", num_cores=NC, num_subcores=NS)


def _tile_table_copy(s, mk_src, mk_dst):
    r0 = s * ROWS_PER_TILE
    pltpu.sync_copy(mk_src(r0, ROWS_PER_TILE), mk_dst(r0, ROWS_PER_TILE))

    @pl.when(s == NS - 1)
    def _():
        pltpu.sync_copy(mk_src(TAIL_OFF, TAIL_ROWS), mk_dst(TAIL_OFF, TAIL_ROWS))


def _worker_row0():
    c = lax.axis_index("c")
    s = lax.axis_index("s")
    wid = c * NS + s
    return c, s, pl.multiple_of(wid * WCH, 8)


HCH = WCH // 2
HPAIRS = HCH // 2


def _seg_sum_phase(p_hbm, src_hbm, dst_hbm, agg_sh, sidx, didx,
                   rows0, rows1, gsem0, gsem1, ssem0, ssem1, h0):
    h0 = pl.multiple_of(h0, 8)
    pltpu.sync_copy(src_hbm.at[pl.ds(h0, HCH)], sidx)
    pltpu.sync_copy(dst_hbm.at[pl.ds(h0, HCH)], didx)
    pltpu.async_copy(p_hbm.at[sidx.at[0]], rows0, gsem0)
    pltpu.async_copy(p_hbm.at[sidx.at[1]], rows1, gsem1)

    def pair(k, carry):
        c0 = 2 * k
        pltpu.make_async_copy(p_hbm.at[sidx.at[0]], rows0, gsem0).wait()
        pltpu.async_copy(rows0, agg_sh.at[didx.at[c0]], ssem0, add=True)
        pltpu.make_async_copy(p_hbm.at[sidx.at[0]], rows1, gsem1).wait()
        pltpu.async_copy(rows1, agg_sh.at[didx.at[c0 + 1]], ssem1, add=True)

        @pl.when(k < HPAIRS - 1)
        def _():
            pltpu.make_async_copy(rows0, agg_sh.at[didx.at[0]], ssem0).wait()
            pltpu.async_copy(p_hbm.at[sidx.at[c0 + 2]], rows0, gsem0)
            pltpu.make_async_copy(rows1, agg_sh.at[didx.at[0]], ssem1).wait()
            pltpu.async_copy(p_hbm.at[sidx.at[c0 + 3]], rows1, gsem1)

        return carry

    lax.fori_loop(0, HPAIRS, pair, 0)
    pltpu.make_async_copy(rows0, agg_sh.at[didx.at[0]], ssem0).wait()
    pltpu.make_async_copy(rows1, agg_sh.at[didx.at[0]], ssem1).wait()


def _seg_sum_body(p_hbm, src_hbm, dst_hbm, z_hbm, agg_out,
                  sidx, didx, rows0, rows1, agg_sh, gsem0, gsem1, ssem0, ssem1):
    c, s, w0 = _worker_row0()

    _tile_table_copy(s, lambda o, n: z_hbm.at[pl.ds(o, n)],
                     lambda o, n: agg_sh.at[pl.ds(o, n)])
    plsc.subcore_barrier()

    _seg_sum_phase(p_hbm, src_hbm, dst_hbm, agg_sh, sidx, didx,
                   rows0, rows1, gsem0, gsem1, ssem0, ssem1, w0)
    _seg_sum_phase(p_hbm, src_hbm, dst_hbm, agg_sh, sidx, didx,
                   rows0, rows1, gsem0, gsem1, ssem0, ssem1, w0 + HCH)
    plsc.subcore_barrier()

    _tile_table_copy(s, lambda o, n: agg_sh.at[pl.ds(o, n)],
                     lambda o, n: agg_out.at[c, pl.ds(o, n)])


_seg_sum = pl.kernel(
    _seg_sum_body,
    out_type=jax.ShapeDtypeStruct((NC, N8, D), jnp.float32),
    mesh=_mesh,
    scratch_types=[
        pltpu.VMEM((HCH, CH), jnp.int32),
        pltpu.VMEM((HCH, CH), jnp.int32),
        pltpu.VMEM((CH, D), jnp.float32),
        pltpu.VMEM((CH, D), jnp.float32),
        pltpu.VMEM_SHARED((N8, D), jnp.float32),
        pltpu.SemaphoreType.DMA,
        pltpu.SemaphoreType.DMA,
        pltpu.SemaphoreType.DMA,
        pltpu.SemaphoreType.DMA,
    ])


def _deg_hist_body(ones_hbm, dst_hbm, z_hbm, deg_out, didx, ones, deg_sh, sem):
    c, s, w0 = _worker_row0()

    _tile_table_copy(s, lambda o, n: z_hbm.at[pl.ds(o, n)],
                     lambda o, n: deg_sh.at[pl.ds(o, n)])
    pltpu.sync_copy(ones_hbm, ones)
    pltpu.sync_copy(dst_hbm.at[pl.ds(w0, WCH)], didx)
    plsc.subcore_barrier()

    def chunk(j, carry):
        pltpu.sync_copy(ones, deg_sh.at[didx.at[j]], add=True)
        return carry

    lax.fori_loop(0, WCH, chunk, 0)
    plsc.subcore_barrier()

    _tile_table_copy(s, lambda o, n: deg_sh.at[pl.ds(o, n)],
                     lambda o, n: deg_out.at[c, pl.ds(o, n)])


_deg_hist = pl.kernel(
    _deg_hist_body,
    out_type=jax.ShapeDtypeStruct((NC, N8, D), jnp.float32),
    mesh=_mesh,
    scratch_types=[
        pltpu.VMEM((WCH, CH), jnp.int32),
        pltpu.VMEM((CH, D), jnp.float32),
        pltpu.VMEM_SHARED((N8, D), jnp.float32),
        pltpu.SemaphoreType.DMA,
    ])


def _final_gather_body(sp_hbm, dp_hbm, src_hbm, dst_hbm, us_out, ud_out,
                       sidx, didx, s0, s1, d0, d1,
                       semA, semB, semC, semD, wsem0, wsem1):
    c, s, w0 = _worker_row0()
    pltpu.sync_copy(src_hbm.at[pl.ds(w0, WCH)], sidx)
    pltpu.sync_copy(dst_hbm.at[pl.ds(w0, WCH)], didx)

    pltpu.async_copy(sp_hbm.at[sidx.at[0]], s0, semA)
    pltpu.async_copy(dp_hbm.at[didx.at[0]], d0, semB)
    pltpu.async_copy(sp_hbm.at[sidx.at[1]], s1, semC)
    pltpu.async_copy(dp_hbm.at[didx.at[1]], d1, semD)

    def pair(k, carry):
        c0 = 2 * k
        off0 = pl.multiple_of((w0 + c0) * CH, 8)
        off1 = pl.multiple_of((w0 + c0 + 1) * CH, 8)
        pltpu.make_async_copy(sp_hbm.at[sidx.at[0]], s0, semA).wait()
        pltpu.async_copy(s0, us_out.at[pl.ds(off0, CH)], wsem0)
        pltpu.make_async_copy(dp_hbm.at[didx.at[0]], d0, semB).wait()
        pltpu.async_copy(d0, ud_out.at[pl.ds(off0, CH)], wsem0)
        pltpu.make_async_copy(sp_hbm.at[sidx.at[0]], s1, semC).wait()
        pltpu.async_copy(s1, us_out.at[pl.ds(off1, CH)], wsem1)
        pltpu.make_async_copy(dp_hbm.at[didx.at[0]], d1, semD).wait()
        pltpu.async_copy(d1, ud_out.at[pl.ds(off1, CH)], wsem1)

        @pl.when(k < PAIRS - 1)
        def _():
            pltpu.make_async_copy(s0, us_out.at[pl.ds(off0, CH)], wsem0).wait()
            pltpu.make_async_copy(d0, ud_out.at[pl.ds(off0, CH)], wsem0).wait()
            pltpu.async_copy(sp_hbm.at[sidx.at[c0 + 2]], s0, semA)
            pltpu.async_copy(dp_hbm.at[didx.at[c0 + 2]], d0, semB)
            pltpu.make_async_copy(s1, us_out.at[pl.ds(off1, CH)], wsem1).wait()
            pltpu.make_async_copy(d1, ud_out.at[pl.ds(off1, CH)], wsem1).wait()
            pltpu.async_copy(sp_hbm.at[sidx.at[c0 + 3]], s1, semC)
            pltpu.async_copy(dp_hbm.at[didx.at[c0 + 3]], d1, semD)

        return carry

    lax.fori_loop(0, PAIRS, pair, 0)
    pltpu.make_async_copy(s0, us_out.at[pl.ds(w0 * CH, CH)], wsem0).wait()
    pltpu.make_async_copy(d0, ud_out.at[pl.ds(w0 * CH, CH)], wsem0).wait()
    pltpu.make_async_copy(s1, us_out.at[pl.ds(w0 * CH, CH)], wsem1).wait()
    pltpu.make_async_copy(d1, ud_out.at[pl.ds(w0 * CH, CH)], wsem1).wait()


_final_gather = pl.kernel(
    _final_gather_body,
    out_type=(jax.ShapeDtypeStruct((EPAD, D), jnp.float32),
              jax.ShapeDtypeStruct((EPAD, D), jnp.float32)),
    mesh=_mesh,
    scratch_types=[
        pltpu.VMEM((WCH, CH), jnp.int32),
        pltpu.VMEM((WCH, CH), jnp.int32),
        pltpu.VMEM((CH, D), jnp.float32),
        pltpu.VMEM((CH, D), jnp.float32),
        pltpu.VMEM((CH, D), jnp.float32),
        pltpu.VMEM((CH, D), jnp.float32),
        pltpu.SemaphoreType.DMA,
        pltpu.SemaphoreType.DMA,
        pltpu.SemaphoreType.DMA,
        pltpu.SemaphoreType.DMA,
        pltpu.SemaphoreType.DMA,
        pltpu.SemaphoreType.DMA,
    ])


def _mm_body(h_ref, w_ref, o_ref):
    o_ref[...] = jnp.dot(h_ref[...], w_ref[...],
                         preferred_element_type=jnp.float32)


_mm = pl.pallas_call(_mm_body, out_shape=jax.ShapeDtypeStruct((N, D), jnp.float32))


def _combine_body(h_ref, agg_ref, deg_ref, ws_ref, b_ref, wn_ref, h_out, p_out):
    a = agg_ref[0, :N] + agg_ref[1, :N]
    dcol = deg_ref[0, :N, 0:1] + deg_ref[1, :N, 0:1]
    inv = 1.0 / jnp.maximum(dcol, 1.0)
    x = jnp.dot(h_ref[...], ws_ref[...], preferred_element_type=jnp.float32)
    x = x + a * inv + b_ref[...]
    hn = jnp.where(x > 0, x, 0.01 * x)
    h_out[...] = hn
    p_out[...] = jnp.dot(hn, wn_ref[...], preferred_element_type=jnp.float32)


_combine = pl.pallas_call(
    _combine_body,
    out_shape=(jax.ShapeDtypeStruct((N, D), jnp.float32),
               jax.ShapeDtypeStruct((N, D), jnp.float32)))


def _combine3_body(h_ref, agg_ref, deg_ref, ws_ref, b_ref, w1a_ref, w1b_ref,
                   bm1_ref, sp_out, dp_out):
    a = agg_ref[0, :N] + agg_ref[1, :N]
    dcol = deg_ref[0, :N, 0:1] + deg_ref[1, :N, 0:1]
    inv = 1.0 / jnp.maximum(dcol, 1.0)
    x = jnp.dot(h_ref[...], ws_ref[...], preferred_element_type=jnp.float32)
    x = x + a * inv + b_ref[...]
    hn = jnp.where(x > 0, x, 0.01 * x)
    sp_out[...] = jnp.dot(hn, w1a_ref[...],
                          preferred_element_type=jnp.float32) + bm1_ref[...]
    dp_out[...] = jnp.dot(hn, w1b_ref[...], preferred_element_type=jnp.float32)


_combine3 = pl.pallas_call(
    _combine3_body,
    out_shape=(jax.ShapeDtypeStruct((N, D), jnp.float32),
               jax.ShapeDtypeStruct((N, D), jnp.float32)))

EBLK = 2560


def _edge_mlp_body(us_ref, ud_ref, eft_ref, w1e_ref, w2_ref, bm2_ref, o_ref):
    ec = lax.dot_general(eft_ref[...], w1e_ref[...], (((0,), (0,)), ((), ())),
                         preferred_element_type=jnp.float32)
    z = us_ref[...] + ud_ref[...] + ec
    z = jnp.maximum(z, 0.0)
    o_ref[...] = lax.dot_general(w2_ref[...], z, (((0,), (1,)), ((), ())),
                                 preferred_element_type=jnp.float32) + bm2_ref[...]


_edge_mlp = pl.pallas_call(
    _edge_mlp_body,
    grid=(E // EBLK,),
    in_specs=[
        pl.BlockSpec((EBLK, D), lambda i: (i, 0)),
        pl.BlockSpec((EBLK, D), lambda i: (i, 0)),
        pl.BlockSpec((DE, EBLK), lambda i: (0, i)),
        pl.BlockSpec((DE, D), lambda i: (0, 0)),
        pl.BlockSpec((D, C), lambda i: (0, 0)),
        pl.BlockSpec((C, 1), lambda i: (0, 0)),
    ],
    out_specs=pl.BlockSpec((C, EBLK), lambda i: (0, i)),
    out_shape=jax.ShapeDtypeStruct((C, E), jnp.float32))


def kernel(node_feats, edge_index, edge_feats,
           Ws0, Wn0, b0, Ws1, Wn1, b1, Ws2, Wn2, b2,
           W1, bm1, W2, bm2):
    src = edge_index[0]
    dst = edge_index[1]
    pad = EPAD - E
    spread = jnp.arange(pad, dtype=jnp.int32) % N
    src2d = jnp.concatenate([src, spread]).reshape(-1, CH)
    dst2d_s = jnp.concatenate(
        [dst, N + (jnp.arange(pad, dtype=jnp.int32) % 8)]).reshape(-1, CH)
    dst2d_g = jnp.concatenate([dst, spread]).reshape(-1, CH)
    z = jnp.zeros((N8, D), jnp.float32)
    ones_ch = jnp.ones((CH, D), jnp.float32)

    deg = _deg_hist(ones_ch, dst2d_s, z)
    p0 = _mm(node_feats, Wn0)
    agg0 = _seg_sum(p0, src2d, dst2d_s, z)
    h1, p1 = _combine(node_feats, agg0, deg, Ws0, b0.reshape(1, D), Wn1)
    agg1 = _seg_sum(p1, src2d, dst2d_s, z)
    h2, p2 = _combine(h1, agg1, deg, Ws1, b1.reshape(1, D), Wn2)
    agg2 = _seg_sum(p2, src2d, dst2d_s, z)
    sp, dp = _combine3(h2, agg2, deg, Ws2, b2.reshape(1, D),
                       W1[:D], W1[D:2 * D], bm1.reshape(1, D))
    us, ud = _final_gather(sp, dp, src2d, dst2d_g)
    logits_t = _edge_mlp(us, ud, edge_feats.T, W1[2 * D:], W2,
                         bm2.reshape(C, 1))
    return logits_t.T


# speedup vs baseline: 1.1265x; 1.1265x over previous
import jax
import jax.numpy as jnp
from jax import lax
from jax.experimental import pallas as pl
from jax.experimental.pallas import tpu as pltpu
from jax.experimental.pallas import tpu_sc as plsc

N = 10000
E = 320000
D = 128
DE = 4
C = 2

NC = 2
NS = 16
NW = NC * NS
CH = 128
WCH = 80
PAIRS = WCH // 2
EPAD = NW * WCH * CH
N8 = N + 8

ROWS_PER_TILE = 624
TAIL_ROWS = N8 - NS * ROWS_PER_TILE
TAIL_OFF = NS * ROWS_PER_TILE

_mesh = plsc.VectorSubcoreMesh(
    core_axis_name="c", subcore_axis_name="s", num_cores=NC, num_subcores=NS)


def _tile_table_copy(s, mk_src, mk_dst):
    r0 = s * ROWS_PER_TILE
    pltpu.sync_copy(mk_src(r0, ROWS_PER_TILE), mk_dst(r0, ROWS_PER_TILE))

    @pl.when(s == NS - 1)
    def _():
        pltpu.sync_copy(mk_src(TAIL_OFF, TAIL_ROWS), mk_dst(TAIL_OFF, TAIL_ROWS))


def _worker_row0():
    c = lax.axis_index("c")
    s = lax.axis_index("s")
    wid = c * NS + s
    return c, s, pl.multiple_of(wid * WCH, 8)


HCH = WCH // 2
HPAIRS = HCH // 2


def _seg_sum_phase(p_hbm, src_hbm, dst_hbm, agg_sh, sidx, didx,
                   rows0, rows1, sem0, sem1, h0):
    h0 = pl.multiple_of(h0, 8)
    pltpu.sync_copy(src_hbm.at[pl.ds(h0, HCH)], sidx)
    pltpu.sync_copy(dst_hbm.at[pl.ds(h0, HCH)], didx)
    pltpu.async_copy(p_hbm.at[sidx.at[0]], rows0, sem0)

    def pair(k, carry):
        c0 = 2 * k
        pltpu.async_copy(p_hbm.at[sidx.at[c0 + 1]], rows1, sem1)
        pltpu.make_async_copy(p_hbm.at[sidx.at[0]], rows0, sem0).wait()
        pltpu.sync_copy(rows0, agg_sh.at[didx.at[c0]], add=True)

        @pl.when(k < HPAIRS - 1)
        def _():
            pltpu.async_copy(p_hbm.at[sidx.at[c0 + 2]], rows0, sem0)

        pltpu.make_async_copy(p_hbm.at[sidx.at[0]], rows1, sem1).wait()
        pltpu.sync_copy(rows1, agg_sh.at[didx.at[c0 + 1]], add=True)
        return carry

    lax.fori_loop(0, HPAIRS, pair, 0)


def _seg_sum_body(p_hbm, src_hbm, dst_hbm, z_hbm, agg_out,
                  sidx, didx, rows0, rows1, agg_sh, sem0, sem1):
    c, s, w0 = _worker_row0()

    _tile_table_copy(s, lambda o, n: z_hbm.at[pl.ds(o, n)],
                     lambda o, n: agg_sh.at[pl.ds(o, n)])
    plsc.subcore_barrier()

    _seg_sum_phase(p_hbm, src_hbm, dst_hbm, agg_sh, sidx, didx,
                   rows0, rows1, sem0, sem1, w0)
    _seg_sum_phase(p_hbm, src_hbm, dst_hbm, agg_sh, sidx, didx,
                   rows0, rows1, sem0, sem1, w0 + HCH)
    plsc.subcore_barrier()

    _tile_table_copy(s, lambda o, n: agg_sh.at[pl.ds(o, n)],
                     lambda o, n: agg_out.at[c, pl.ds(o, n)])


_seg_sum = pl.kernel(
    _seg_sum_body,
    out_type=jax.ShapeDtypeStruct((NC, N8, D), jnp.float32),
    mesh=_mesh,
    scratch_types=[
        pltpu.VMEM((HCH, CH), jnp.int32),
        pltpu.VMEM((HCH, CH), jnp.int32),
        pltpu.VMEM((CH, D), jnp.float32),
        pltpu.VMEM((CH, D), jnp.float32),
        pltpu.VMEM_SHARED((N8, D), jnp.float32),
        pltpu.SemaphoreType.DMA,
        pltpu.SemaphoreType.DMA,
    ])


def _deg_hist_body(ones_hbm, dst_hbm, z_hbm, deg_out, didx, ones, deg_sh, sem):
    c, s, w0 = _worker_row0()

    _tile_table_copy(s, lambda o, n: z_hbm.at[pl.ds(o, n)],
                     lambda o, n: deg_sh.at[pl.ds(o, n)])
    pltpu.sync_copy(ones_hbm, ones)
    pltpu.sync_copy(dst_hbm.at[pl.ds(w0, WCH)], didx)
    plsc.subcore_barrier()

    def chunk(j, carry):
        pltpu.sync_copy(ones, deg_sh.at[didx.at[j]], add=True)
        return carry

    lax.fori_loop(0, WCH, chunk, 0)
    plsc.subcore_barrier()

    _tile_table_copy(s, lambda o, n: deg_sh.at[pl.ds(o, n)],
                     lambda o, n: deg_out.at[c, pl.ds(o, n)])


_deg_hist = pl.kernel(
    _deg_hist_body,
    out_type=jax.ShapeDtypeStruct((NC, N8, D), jnp.float32),
    mesh=_mesh,
    scratch_types=[
        pltpu.VMEM((WCH, CH), jnp.int32),
        pltpu.VMEM((CH, D), jnp.float32),
        pltpu.VMEM_SHARED((N8, D), jnp.float32),
        pltpu.SemaphoreType.DMA,
    ])


def _final_gather_body(sp_hbm, dp_hbm, src_hbm, dst_hbm, us_out, ud_out,
                       sidx, didx, s0, s1, d0, d1, semA, semB, semC, semD):
    c, s, w0 = _worker_row0()
    pltpu.sync_copy(src_hbm.at[pl.ds(w0, WCH)], sidx)
    pltpu.sync_copy(dst_hbm.at[pl.ds(w0, WCH)], didx)

    pltpu.async_copy(sp_hbm.at[sidx.at[0]], s0, semA)
    pltpu.async_copy(dp_hbm.at[didx.at[0]], d0, semB)

    def pair(k, carry):
        c0 = 2 * k
        off0 = pl.multiple_of((w0 + c0) * CH, 8)
        off1 = pl.multiple_of((w0 + c0 + 1) * CH, 8)
        pltpu.async_copy(sp_hbm.at[sidx.at[c0 + 1]], s1, semC)
        pltpu.async_copy(dp_hbm.at[didx.at[c0 + 1]], d1, semD)
        pltpu.make_async_copy(sp_hbm.at[sidx.at[0]], s0, semA).wait()
        pltpu.make_async_copy(dp_hbm.at[didx.at[0]], d0, semB).wait()
        pltpu.sync_copy(s0, us_out.at[pl.ds(off0, CH)])
        pltpu.sync_copy(d0, ud_out.at[pl.ds(off0, CH)])

        @pl.when(k < PAIRS - 1)
        def _():
            pltpu.async_copy(sp_hbm.at[sidx.at[c0 + 2]], s0, semA)
            pltpu.async_copy(dp_hbm.at[didx.at[c0 + 2]], d0, semB)

        pltpu.make_async_copy(sp_hbm.at[sidx.at[0]], s1, semC).wait()
        pltpu.make_async_copy(dp_hbm.at[didx.at[0]], d1, semD).wait()
        pltpu.sync_copy(s1, us_out.at[pl.ds(off1, CH)])
        pltpu.sync_copy(d1, ud_out.at[pl.ds(off1, CH)])
        return carry

    lax.fori_loop(0, PAIRS, pair, 0)


_final_gather = pl.kernel(
    _final_gather_body,
    out_type=(jax.ShapeDtypeStruct((EPAD, D), jnp.float32),
              jax.ShapeDtypeStruct((EPAD, D), jnp.float32)),
    mesh=_mesh,
    scratch_types=[
        pltpu.VMEM((WCH, CH), jnp.int32),
        pltpu.VMEM((WCH, CH), jnp.int32),
        pltpu.VMEM((CH, D), jnp.float32),
        pltpu.VMEM((CH, D), jnp.float32),
        pltpu.VMEM((CH, D), jnp.float32),
        pltpu.VMEM((CH, D), jnp.float32),
        pltpu.SemaphoreType.DMA,
        pltpu.SemaphoreType.DMA,
        pltpu.SemaphoreType.DMA,
        pltpu.SemaphoreType.DMA,
    ])


def _mm_body(h_ref, w_ref, o_ref):
    o_ref[...] = jnp.dot(h_ref[...], w_ref[...],
                         preferred_element_type=jnp.float32)


_mm = pl.pallas_call(_mm_body, out_shape=jax.ShapeDtypeStruct((N, D), jnp.float32))


def _combine_body(h_ref, agg_ref, deg_ref, ws_ref, b_ref, wn_ref, h_out, p_out):
    a = agg_ref[0, :N] + agg_ref[1, :N]
    dcol = deg_ref[0, :N, 0:1] + deg_ref[1, :N, 0:1]
    inv = 1.0 / jnp.maximum(dcol, 1.0)
    x = jnp.dot(h_ref[...], ws_ref[...], preferred_element_type=jnp.float32)
    x = x + a * inv + b_ref[...]
    hn = jnp.where(x > 0, x, 0.01 * x)
    h_out[...] = hn
    p_out[...] = jnp.dot(hn, wn_ref[...], preferred_element_type=jnp.float32)


_combine = pl.pallas_call(
    _combine_body,
    out_shape=(jax.ShapeDtypeStruct((N, D), jnp.float32),
               jax.ShapeDtypeStruct((N, D), jnp.float32)))


def _combine3_body(h_ref, agg_ref, deg_ref, ws_ref, b_ref, w1a_ref, w1b_ref,
                   bm1_ref, sp_out, dp_out):
    a = agg_ref[0, :N] + agg_ref[1, :N]
    dcol = deg_ref[0, :N, 0:1] + deg_ref[1, :N, 0:1]
    inv = 1.0 / jnp.maximum(dcol, 1.0)
    x = jnp.dot(h_ref[...], ws_ref[...], preferred_element_type=jnp.float32)
    x = x + a * inv + b_ref[...]
    hn = jnp.where(x > 0, x, 0.01 * x)
    sp_out[...] = jnp.dot(hn, w1a_ref[...],
                          preferred_element_type=jnp.float32) + bm1_ref[...]
    dp_out[...] = jnp.dot(hn, w1b_ref[...], preferred_element_type=jnp.float32)


_combine3 = pl.pallas_call(
    _combine3_body,
    out_shape=(jax.ShapeDtypeStruct((N, D), jnp.float32),
               jax.ShapeDtypeStruct((N, D), jnp.float32)))

EBLK = 2560


def _edge_mlp_body(us_ref, ud_ref, eft_ref, w1e_ref, w2_ref, bm2_ref, o_ref):
    ec = lax.dot_general(eft_ref[...], w1e_ref[...], (((0,), (0,)), ((), ())),
                         preferred_element_type=jnp.float32)
    z = us_ref[...] + ud_ref[...] + ec
    z = jnp.maximum(z, 0.0)
    o_ref[...] = lax.dot_general(w2_ref[...], z, (((0,), (1,)), ((), ())),
                                 preferred_element_type=jnp.float32) + bm2_ref[...]


_edge_mlp = pl.pallas_call(
    _edge_mlp_body,
    grid=(E // EBLK,),
    in_specs=[
        pl.BlockSpec((EBLK, D), lambda i: (i, 0)),
        pl.BlockSpec((EBLK, D), lambda i: (i, 0)),
        pl.BlockSpec((DE, EBLK), lambda i: (0, i)),
        pl.BlockSpec((DE, D), lambda i: (0, 0)),
        pl.BlockSpec((D, C), lambda i: (0, 0)),
        pl.BlockSpec((C, 1), lambda i: (0, 0)),
    ],
    out_specs=pl.BlockSpec((C, EBLK), lambda i: (0, i)),
    out_shape=jax.ShapeDtypeStruct((C, E), jnp.float32))


def kernel(node_feats, edge_index, edge_feats,
           Ws0, Wn0, b0, Ws1, Wn1, b1, Ws2, Wn2, b2,
           W1, bm1, W2, bm2):
    src = edge_index[0]
    dst = edge_index[1]
    pad = EPAD - E
    spread = jnp.arange(pad, dtype=jnp.int32) % N
    src2d = jnp.concatenate([src, spread]).reshape(-1, CH)
    dst2d_s = jnp.concatenate(
        [dst, N + (jnp.arange(pad, dtype=jnp.int32) % 8)]).reshape(-1, CH)
    dst2d_g = jnp.concatenate([dst, spread]).reshape(-1, CH)
    z = jnp.zeros((N8, D), jnp.float32)
    ones_ch = jnp.ones((CH, D), jnp.float32)

    deg = _deg_hist(ones_ch, dst2d_s, z)
    p0 = _mm(node_feats, Wn0)
    agg0 = _seg_sum(p0, src2d, dst2d_s, z)
    h1, p1 = _combine(node_feats, agg0, deg, Ws0, b0.reshape(1, D), Wn1)
    agg1 = _seg_sum(p1, src2d, dst2d_s, z)
    h2, p2 = _combine(h1, agg1, deg, Ws1, b1.reshape(1, D), Wn2)
    agg2 = _seg_sum(p2, src2d, dst2d_s, z)
    sp, dp = _combine3(h2, agg2, deg, Ws2, b2.reshape(1, D),
                       W1[:D], W1[D:2 * D], bm1.reshape(1, D))
    us, ud = _final_gather(sp, dp, src2d, dst2d_g)
    logits_t = _edge_mlp(us, ud, edge_feats.T, W1[2 * D:], W2,
                         bm2.reshape(C, 1))
    return logits_t.T


# speedup vs baseline: 1.1478x; 1.0189x over previous
import jax
import jax.numpy as jnp
from jax import lax
from jax.experimental import pallas as pl
from jax.experimental.pallas import tpu as pltpu
from jax.experimental.pallas import tpu_sc as plsc

N = 10000
E = 320000
D = 128
DE = 4
C = 2

NC = 2
NS = 16
NW = NC * NS
CH = 128
WCH = 80
PAIRS = WCH // 2
EPAD = NW * WCH * CH
N8 = N + 8

ROWS_PER_TILE = 624
TAIL_ROWS = N8 - NS * ROWS_PER_TILE
TAIL_OFF = NS * ROWS_PER_TILE

_mesh = plsc.VectorSubcoreMesh(
    core_axis_name="c", subcore_axis_name="s", num_cores=NC, num_subcores=NS)


def _tile_table_copy(s, mk_src, mk_dst):
    r0 = s * ROWS_PER_TILE
    pltpu.sync_copy(mk_src(r0, ROWS_PER_TILE), mk_dst(r0, ROWS_PER_TILE))

    @pl.when(s == NS - 1)
    def _():
        pltpu.sync_copy(mk_src(TAIL_OFF, TAIL_ROWS), mk_dst(TAIL_OFF, TAIL_ROWS))


def _worker_row0():
    c = lax.axis_index("c")
    s = lax.axis_index("s")
    wid = c * NS + s
    return c, s, pl.multiple_of(wid * WCH, 8)


HCH = WCH // 2
HPAIRS = HCH // 2


def _seg_sum_phase(p_hbm, src_hbm, dst_hbm, agg_sh, sidx, didx,
                   rows0, rows1, sem0, sem1, h0):
    h0 = pl.multiple_of(h0, 8)
    pltpu.sync_copy(src_hbm.at[pl.ds(h0, HCH)], sidx)
    pltpu.sync_copy(dst_hbm.at[pl.ds(h0, HCH)], didx)
    pltpu.async_copy(p_hbm.at[sidx.at[0]], rows0, sem0)

    def pair(k, carry):
        c0 = 2 * k
        pltpu.async_copy(p_hbm.at[sidx.at[c0 + 1]], rows1, sem1)
        pltpu.make_async_copy(p_hbm.at[sidx.at[0]], rows0, sem0).wait()
        pltpu.sync_copy(rows0, agg_sh.at[didx.at[c0]], add=True)

        @pl.when(k < HPAIRS - 1)
        def _():
            pltpu.async_copy(p_hbm.at[sidx.at[c0 + 2]], rows0, sem0)

        pltpu.make_async_copy(p_hbm.at[sidx.at[0]], rows1, sem1).wait()
        pltpu.sync_copy(rows1, agg_sh.at[didx.at[c0 + 1]], add=True)
        return carry

    lax.fori_loop(0, HPAIRS, pair, 0)


def _seg_sum_body(p_hbm, src_hbm, dst_hbm, z_hbm, agg_out,
                  sidx, didx, rows0, rows1, agg_sh, sem0, sem1):
    c, s, w0 = _worker_row0()

    _tile_table_copy(s, lambda o, n: z_hbm.at[pl.ds(o, n)],
                     lambda o, n: agg_sh.at[pl.ds(o, n)])
    plsc.subcore_barrier()

    _seg_sum_phase(p_hbm, src_hbm, dst_hbm, agg_sh, sidx, didx,
                   rows0, rows1, sem0, sem1, w0)
    _seg_sum_phase(p_hbm, src_hbm, dst_hbm, agg_sh, sidx, didx,
                   rows0, rows1, sem0, sem1, w0 + HCH)
    plsc.subcore_barrier()

    _tile_table_copy(s, lambda o, n: agg_sh.at[pl.ds(o, n)],
                     lambda o, n: agg_out.at[c, pl.ds(o, n)])


_seg_sum = pl.kernel(
    _seg_sum_body,
    out_type=jax.ShapeDtypeStruct((NC, N8, D), jnp.float32),
    mesh=_mesh,
    scratch_types=[
        pltpu.VMEM((HCH, CH), jnp.int32),
        pltpu.VMEM((HCH, CH), jnp.int32),
        pltpu.VMEM((CH, D), jnp.float32),
        pltpu.VMEM((CH, D), jnp.float32),
        pltpu.VMEM_SHARED((N8, D), jnp.float32),
        pltpu.SemaphoreType.DMA,
        pltpu.SemaphoreType.DMA,
    ])


def _deg_hist_body(ones_hbm, dst_hbm, z_hbm, deg_out, didx, ones, deg_sh, sem):
    c, s, w0 = _worker_row0()

    _tile_table_copy(s, lambda o, n: z_hbm.at[pl.ds(o, n)],
                     lambda o, n: deg_sh.at[pl.ds(o, n)])
    pltpu.sync_copy(ones_hbm, ones)
    pltpu.sync_copy(dst_hbm.at[pl.ds(w0, WCH)], didx)
    plsc.subcore_barrier()

    def chunk(j, carry):
        pltpu.sync_copy(ones, deg_sh.at[didx.at[j]], add=True)
        return carry

    lax.fori_loop(0, WCH, chunk, 0)
    plsc.subcore_barrier()

    _tile_table_copy(s, lambda o, n: deg_sh.at[pl.ds(o, n)],
                     lambda o, n: deg_out.at[c, pl.ds(o, n)])


_deg_hist = pl.kernel(
    _deg_hist_body,
    out_type=jax.ShapeDtypeStruct((NC, N8, D), jnp.float32),
    mesh=_mesh,
    scratch_types=[
        pltpu.VMEM((WCH, CH), jnp.int32),
        pltpu.VMEM((CH, D), jnp.float32),
        pltpu.VMEM_SHARED((N8, D), jnp.float32),
        pltpu.SemaphoreType.DMA,
    ])


def _final_gather_body(half, sp_hbm, dp_hbm, src_hbm, dst_hbm, us_out, ud_out,
                       sidx, didx, s0, s1, d0, d1, semA, semB, semC, semD):
    c = lax.axis_index("c")
    s = lax.axis_index("s")
    wid = c * NS + s
    w0 = pl.multiple_of(half * (NW * HCH) + wid * HCH, 8)
    wo = pl.multiple_of(wid * HCH, 8)
    pltpu.sync_copy(src_hbm.at[pl.ds(w0, HCH)], sidx)
    pltpu.sync_copy(dst_hbm.at[pl.ds(w0, HCH)], didx)

    pltpu.async_copy(sp_hbm.at[sidx.at[0]], s0, semA)
    pltpu.async_copy(dp_hbm.at[didx.at[0]], d0, semB)

    def pair(k, carry):
        c0 = 2 * k
        off0 = pl.multiple_of((wo + c0) * CH, 8)
        off1 = pl.multiple_of((wo + c0 + 1) * CH, 8)
        pltpu.async_copy(sp_hbm.at[sidx.at[c0 + 1]], s1, semC)
        pltpu.async_copy(dp_hbm.at[didx.at[c0 + 1]], d1, semD)
        pltpu.make_async_copy(sp_hbm.at[sidx.at[0]], s0, semA).wait()
        pltpu.make_async_copy(dp_hbm.at[didx.at[0]], d0, semB).wait()
        pltpu.sync_copy(s0, us_out.at[pl.ds(off0, CH)])
        pltpu.sync_copy(d0, ud_out.at[pl.ds(off0, CH)])

        @pl.when(k < HPAIRS - 1)
        def _():
            pltpu.async_copy(sp_hbm.at[sidx.at[c0 + 2]], s0, semA)
            pltpu.async_copy(dp_hbm.at[didx.at[c0 + 2]], d0, semB)

        pltpu.make_async_copy(sp_hbm.at[sidx.at[0]], s1, semC).wait()
        pltpu.make_async_copy(dp_hbm.at[didx.at[0]], d1, semD).wait()
        pltpu.sync_copy(s1, us_out.at[pl.ds(off1, CH)])
        pltpu.sync_copy(d1, ud_out.at[pl.ds(off1, CH)])
        return carry

    lax.fori_loop(0, HPAIRS, pair, 0)


import functools as _ft

EHALF = EPAD // 2


def _make_final_gather(half):
    return pl.kernel(
        _ft.partial(_final_gather_body, half),
        out_type=(jax.ShapeDtypeStruct((EHALF, D), jnp.float32),
                  jax.ShapeDtypeStruct((EHALF, D), jnp.float32)),
        mesh=_mesh,
        scratch_types=[
            pltpu.VMEM((HCH, CH), jnp.int32),
            pltpu.VMEM((HCH, CH), jnp.int32),
            pltpu.VMEM((CH, D), jnp.float32),
            pltpu.VMEM((CH, D), jnp.float32),
            pltpu.VMEM((CH, D), jnp.float32),
            pltpu.VMEM((CH, D), jnp.float32),
            pltpu.SemaphoreType.DMA,
            pltpu.SemaphoreType.DMA,
            pltpu.SemaphoreType.DMA,
            pltpu.SemaphoreType.DMA,
        ])


_final_gather0 = _make_final_gather(0)
_final_gather1 = _make_final_gather(1)


def _mm_body(h_ref, w_ref, o_ref):
    o_ref[...] = jnp.dot(h_ref[...], w_ref[...],
                         preferred_element_type=jnp.float32)


_mm = pl.pallas_call(_mm_body, out_shape=jax.ShapeDtypeStruct((N, D), jnp.float32))


def _combine_body(h_ref, agg_ref, deg_ref, ws_ref, b_ref, wn_ref, h_out, p_out):
    a = agg_ref[0, :N] + agg_ref[1, :N]
    dcol = deg_ref[0, :N, 0:1] + deg_ref[1, :N, 0:1]
    inv = 1.0 / jnp.maximum(dcol, 1.0)
    x = jnp.dot(h_ref[...], ws_ref[...], preferred_element_type=jnp.float32)
    x = x + a * inv + b_ref[...]
    hn = jnp.where(x > 0, x, 0.01 * x)
    h_out[...] = hn
    p_out[...] = jnp.dot(hn, wn_ref[...], preferred_element_type=jnp.float32)


_combine = pl.pallas_call(
    _combine_body,
    out_shape=(jax.ShapeDtypeStruct((N, D), jnp.float32),
               jax.ShapeDtypeStruct((N, D), jnp.float32)))


def _combine3_body(h_ref, agg_ref, deg_ref, ws_ref, b_ref, w1a_ref, w1b_ref,
                   bm1_ref, sp_out, dp_out):
    a = agg_ref[0, :N] + agg_ref[1, :N]
    dcol = deg_ref[0, :N, 0:1] + deg_ref[1, :N, 0:1]
    inv = 1.0 / jnp.maximum(dcol, 1.0)
    x = jnp.dot(h_ref[...], ws_ref[...], preferred_element_type=jnp.float32)
    x = x + a * inv + b_ref[...]
    hn = jnp.where(x > 0, x, 0.01 * x)
    sp_out[...] = jnp.dot(hn, w1a_ref[...],
                          preferred_element_type=jnp.float32) + bm1_ref[...]
    dp_out[...] = jnp.dot(hn, w1b_ref[...], preferred_element_type=jnp.float32)


_combine3 = pl.pallas_call(
    _combine3_body,
    out_shape=(jax.ShapeDtypeStruct((N, D), jnp.float32),
               jax.ShapeDtypeStruct((N, D), jnp.float32)))

EBLK = 2560


def _edge_mlp_body(us_ref, ud_ref, eft_ref, w1e_ref, w2_ref, bm2_ref, o_ref):
    ec = lax.dot_general(eft_ref[...], w1e_ref[...], (((0,), (0,)), ((), ())),
                         preferred_element_type=jnp.float32)
    z = us_ref[...] + ud_ref[...] + ec
    z = jnp.maximum(z, 0.0)
    o_ref[...] = lax.dot_general(w2_ref[...], z, (((0,), (1,)), ((), ())),
                                 preferred_element_type=jnp.float32) + bm2_ref[...]


EH0 = EPAD // 2
EH1 = E - EH0
_EOFF_BLKS = EH0 // EBLK


def _make_edge_mlp(half):
    nblk = (EH0 if half == 0 else EH1) // EBLK
    off = _EOFF_BLKS if half else 0
    return pl.pallas_call(
        _edge_mlp_body,
        grid=(nblk,),
        in_specs=[
            pl.BlockSpec((EBLK, D), lambda i: (i, 0)),
            pl.BlockSpec((EBLK, D), lambda i: (i, 0)),
            pl.BlockSpec((DE, EBLK), lambda i, _o=off: (0, _o + i)),
            pl.BlockSpec((DE, D), lambda i: (0, 0)),
            pl.BlockSpec((D, C), lambda i: (0, 0)),
            pl.BlockSpec((C, 1), lambda i: (0, 0)),
        ],
        out_specs=pl.BlockSpec((C, EBLK), lambda i: (0, i)),
        out_shape=jax.ShapeDtypeStruct((C, nblk * EBLK), jnp.float32))


_edge_mlp0 = _make_edge_mlp(0)
_edge_mlp1 = _make_edge_mlp(1)


def kernel(node_feats, edge_index, edge_feats,
           Ws0, Wn0, b0, Ws1, Wn1, b1, Ws2, Wn2, b2,
           W1, bm1, W2, bm2):
    src = edge_index[0]
    dst = edge_index[1]
    pad = EPAD - E
    spread = jnp.arange(pad, dtype=jnp.int32) % N
    src2d = jnp.concatenate([src, spread]).reshape(-1, CH)
    dst2d_s = jnp.concatenate(
        [dst, N + (jnp.arange(pad, dtype=jnp.int32) % 8)]).reshape(-1, CH)
    dst2d_g = jnp.concatenate([dst, spread]).reshape(-1, CH)
    z = jnp.zeros((N8, D), jnp.float32)
    ones_ch = jnp.ones((CH, D), jnp.float32)

    deg = _deg_hist(ones_ch, dst2d_s, z)
    p0 = _mm(node_feats, Wn0)
    agg0 = _seg_sum(p0, src2d, dst2d_s, z)
    h1, p1 = _combine(node_feats, agg0, deg, Ws0, b0.reshape(1, D), Wn1)
    agg1 = _seg_sum(p1, src2d, dst2d_s, z)
    h2, p2 = _combine(h1, agg1, deg, Ws1, b1.reshape(1, D), Wn2)
    agg2 = _seg_sum(p2, src2d, dst2d_s, z)
    sp, dp = _combine3(h2, agg2, deg, Ws2, b2.reshape(1, D),
                       W1[:D], W1[D:2 * D], bm1.reshape(1, D))
    us0, ud0 = _final_gather0(sp, dp, src2d, dst2d_g)
    us1, ud1 = _final_gather1(sp, dp, src2d, dst2d_g)
    eft = edge_feats.T
    w1e = W1[2 * D:]
    bm2c = bm2.reshape(C, 1)
    lt0 = _edge_mlp0(us0, ud0, eft, w1e, W2, bm2c)
    lt1 = _edge_mlp1(us1, ud1, eft, w1e, W2, bm2c)
    return jnp.concatenate([lt0, lt1], axis=1).T
